# Initial kernel scaffold; baseline (speedup 1.0000x reference)
import jax
import jax.numpy as jnp
from jax.experimental import pallas as pl


def kernel(x, node_type, edge_index, edge_type, node_emb, W1, root1, b1, W2, root2, b2, W3, root3, b3):
    raise NotImplementedError("write your pallas kernel here")


# speedup vs baseline: 2.4777x; 2.4777x over previous
import functools

import jax
import jax.numpy as jnp
from jax.experimental import pallas as pl

_R = 4
_BLK = 1000


def _layer_kernel(h_ref, m_ref, root_ref, w_ref, b_ref, o_ref):
    acc = jnp.dot(h_ref[...], root_ref[...],
                  preferred_element_type=jnp.float32) + b_ref[...]
    for r in range(_R):
        acc = acc + jnp.dot(m_ref[r], w_ref[r],
                            preferred_element_type=jnp.float32)
    o_ref[...] = jnp.maximum(acc, 0.0)


def _layer(h, means, root, w, b):
    n, cin = h.shape
    cout = w.shape[2]
    return pl.pallas_call(
        _layer_kernel,
        grid=(n // _BLK,),
        in_specs=[
            pl.BlockSpec((_BLK, cin), lambda i: (i, 0)),
            pl.BlockSpec((_R, _BLK, cin), lambda i: (0, i, 0)),
            pl.BlockSpec((cin, cout), lambda i: (0, 0)),
            pl.BlockSpec((_R, cin, cout), lambda i: (0, 0, 0)),
            pl.BlockSpec((1, cout), lambda i: (0, 0)),
        ],
        out_specs=pl.BlockSpec((_BLK, cout), lambda i: (i, 0)),
        out_shape=jax.ShapeDtypeStruct((n, cout), jnp.float32),
    )(h, means, root, w, b.reshape(1, cout))


def _aggregate(h, src, seg, n):
    msgs = jnp.take(h, src, axis=0)
    summed = jax.ops.segment_sum(msgs, seg, num_segments=n * _R)
    cnt = jax.ops.segment_sum(jnp.ones((seg.shape[0],), jnp.float32), seg,
                              num_segments=n * _R)
    means = summed.reshape(n, _R, -1) / jnp.clip(cnt, 1.0, None).reshape(n, _R, 1)
    return means.transpose(1, 0, 2)


def kernel(x, node_type, edge_index, edge_type, node_emb,
           W1, root1, b1, W2, root2, b2, W3, root3, b3):
    n = x.shape[0]
    h = jnp.concatenate([x, jnp.take(node_emb, node_type, axis=0)], axis=-1)
    src, dst = edge_index[0], edge_index[1]
    seg = dst * _R + edge_type
    for (w, root, b) in ((W1, root1, b1), (W2, root2, b2), (W3, root3, b3)):
        means = _aggregate(h, src, seg, n)
        h = _layer(h, means, root, w, b)
    return h
